# baseline (speedup 1.0000x reference)
import functools

import jax
import jax.numpy as jnp
import numpy as np
from jax import lax
from jax.experimental import pallas as pl
from jax.experimental.pallas import tpu as pltpu
from jax.experimental.pallas import tpu_sc as plsc

_T = 1000
_betas = np.linspace(0.0001, 0.02, _T, dtype=np.float64)
_acp = np.cumprod(1.0 - _betas, axis=0)
_A_NP = np.sqrt(_acp).astype(np.float32)
_C_NP = np.sqrt((1.0 - _acp).astype(np.float32))

_PAD = 1024
_A_TABLE = jnp.asarray(np.pad(_A_NP, (0, _PAD - _T)))
_C_TABLE = jnp.asarray(np.pad(_C_NP, (0, _PAD - _T)))

_B = 32
_F = 3 * 224 * 224
_ROWS = _F // 128


@functools.partial(
    pl.kernel,
    out_type=jax.ShapeDtypeStruct((2, _B), jnp.float32),
    mesh=plsc.VectorSubcoreMesh(core_axis_name="c", subcore_axis_name="s"),
    scratch_types=[
        pltpu.VMEM((_B,), jnp.int32),
        pltpu.VMEM((_B,), jnp.float32),
        pltpu.VMEM((_B,), jnp.float32),
        pltpu.SemaphoreType.DMA,
    ],
)
def _sc_coef(a_hbm, c_hbm, t_hbm, coef_out, t_v, ao_v, co_v, sem):
    wid = lax.axis_index("s") * 2 + lax.axis_index("c")

    @pl.when(wid == 0)
    def _():
        pltpu.sync_copy(t_hbm, t_v)
        pltpu.async_copy(a_hbm.at[t_v], ao_v, sem).wait()
        pltpu.async_copy(c_hbm.at[t_v], co_v, sem).wait()
        pltpu.sync_copy(ao_v, coef_out.at[0])
        pltpu.sync_copy(co_v, coef_out.at[1])


_BB = 8


def _tc_body(coef_ref, x_ref, n_ref, o_ref):
    i = pl.program_id(0)
    for k in range(_BB):
        b = i * _BB + k
        o_ref[k] = x_ref[k] * coef_ref[0, b] + n_ref[k] * coef_ref[1, b]


_tc_call = pl.pallas_call(
    _tc_body,
    grid=(_B // _BB,),
    in_specs=[
        pl.BlockSpec(memory_space=pltpu.SMEM),
        pl.BlockSpec((_BB, 3, 224, 224), lambda i: (i, 0, 0, 0)),
        pl.BlockSpec((_BB, 3, 224, 224), lambda i: (i, 0, 0, 0)),
    ],
    out_specs=pl.BlockSpec((_BB, 3, 224, 224), lambda i: (i, 0, 0, 0)),
    out_shape=jax.ShapeDtypeStruct((_B, 3, 224, 224), jnp.float32),
    compiler_params=pltpu.CompilerParams(dimension_semantics=("parallel",)),
)


@jax.jit
def kernel(x_0, t, noise):
    coef = _sc_coef(_A_TABLE, _C_TABLE, t)
    return _tc_call(coef, x_0, noise)


# speedup vs baseline: 1.0189x; 1.0189x over previous
import functools

import jax
import jax.numpy as jnp
import numpy as np
from jax import lax
from jax.experimental import pallas as pl
from jax.experimental.pallas import tpu as pltpu
from jax.experimental.pallas import tpu_sc as plsc

_T = 1000
_betas = np.linspace(0.0001, 0.02, _T, dtype=np.float64)
_acp = np.cumprod(1.0 - _betas, axis=0)
_A_NP = np.sqrt(_acp).astype(np.float32)
_C_NP = np.sqrt((1.0 - _acp).astype(np.float32))

_PAD = 1024
_TABLE = np.concatenate(
    [np.pad(_A_NP, (0, _PAD - _T)), np.pad(_C_NP, (0, _PAD - _T))]
)

_B = 32


@functools.partial(
    pl.kernel,
    out_type=jax.ShapeDtypeStruct((2 * _B,), jnp.float32),
    mesh=plsc.VectorSubcoreMesh(core_axis_name="c", subcore_axis_name="s"),
    scratch_types=[
        pltpu.VMEM((2 * _B,), jnp.int32),
        pltpu.VMEM((2 * _B,), jnp.float32),
        pltpu.SemaphoreType.DMA,
    ],
)
def _sc_coef(tab_hbm, t_hbm, coef_out, idx_v, out_v, sem):
    wid = lax.axis_index("s") * 2 + lax.axis_index("c")

    @pl.when(wid == 0)
    def _():
        pltpu.sync_copy(t_hbm, idx_v.at[pl.ds(0, _B)])
        for h in range(_B // 16):
            idx_v[pl.ds(_B + 16 * h, 16)] = idx_v[pl.ds(16 * h, 16)] + _PAD
        pltpu.async_copy(tab_hbm.at[idx_v], out_v, sem).wait()
        pltpu.sync_copy(out_v, coef_out)


_BB = 4


def _tc_body(coef_ref, x_ref, n_ref, o_ref):
    i = pl.program_id(0)
    for k in range(_BB):
        b = i * _BB + k
        o_ref[k] = x_ref[k] * coef_ref[b] + n_ref[k] * coef_ref[_B + b]


_tc_call = pl.pallas_call(
    _tc_body,
    grid=(_B // _BB,),
    in_specs=[
        pl.BlockSpec(memory_space=pltpu.SMEM),
        pl.BlockSpec((_BB, 3, 224, 224), lambda i: (i, 0, 0, 0)),
        pl.BlockSpec((_BB, 3, 224, 224), lambda i: (i, 0, 0, 0)),
    ],
    out_specs=pl.BlockSpec((_BB, 3, 224, 224), lambda i: (i, 0, 0, 0)),
    out_shape=jax.ShapeDtypeStruct((_B, 3, 224, 224), jnp.float32),
    compiler_params=pltpu.CompilerParams(dimension_semantics=("parallel",)),
)


@jax.jit
def kernel(x_0, t, noise):
    coef = _sc_coef(_TABLE, t)
    return _tc_call(coef, x_0, noise)
